# Initial kernel scaffold; baseline (speedup 1.0000x reference)
import jax
import jax.numpy as jnp
from jax.experimental import pallas as pl


def kernel(x, w, src, dst, anchors):
    raise NotImplementedError("write your pallas kernel here")


# speedup vs baseline: 79.0891x; 79.0891x over previous
import functools

import jax
import jax.numpy as jnp
from jax import lax
from jax.experimental import pallas as pl
from jax.experimental.pallas import tpu as pltpu
from jax.experimental.pallas import tpu_sc as plsc

NC = 2
NS = 16
L = 16
NW = NC * NS
CE = 2048
CR = CE // 128


def _agg_body(x_hbm, w_hbm, src_hbm, dst_hbm, anch_hbm, ones_hbm, zeros_hbm,
              s_out, cnt_out, h_scr,
              h_vm, src_vm, w_vm, dst_vm, m_vm, anch_vm, xa_vm, ones_vm,
              s_sh, cnt_sh,
              *, nacc, sl, rpw, chunks, arows):
    c = lax.axis_index("c")
    s = lax.axis_index("s")
    wid = s * NC + c

    off = s * sl
    pltpu.sync_copy(zeros_hbm.at[pl.ds(off, sl)], s_sh.at[pl.ds(off, sl)])
    pltpu.sync_copy(zeros_hbm.at[pl.ds(off, sl)], cnt_sh.at[pl.ds(off, sl)])
    pltpu.sync_copy(ones_hbm, ones_vm)
    plsc.subcore_barrier()

    @pl.when(s == 0)
    def _build_h():
        pltpu.sync_copy(anch_hbm, anch_vm)

        def set_ones(j, carry):
            pltpu.sync_copy(ones_vm.at[0], s_sh.at[anch_vm.at[j]])
            return carry
        lax.fori_loop(0, arows, set_ones, 0)

        def add_x(j, carry):
            pltpu.sync_copy(x_hbm.at[anch_vm.at[j]], xa_vm.at[0])
            pltpu.sync_copy(xa_vm.at[0], s_sh.at[anch_vm.at[j]], add=True)
            return carry
        lax.fori_loop(0, arows, add_x, 0)

        pltpu.sync_copy(s_sh, h_scr.at[c])

    plsc.subcore_barrier()

    pltpu.sync_copy(zeros_hbm.at[pl.ds(off, sl)], s_sh.at[pl.ds(off, sl)])
    pltpu.sync_copy(h_scr.at[c], h_vm)
    plsc.subcore_barrier()

    ebase = wid * rpw * 128
    rbase = wid * rpw

    def chunk_body(k, carry):
        er = ebase + k * CE
        rr = rbase + k * CR
        pltpu.sync_copy(src_hbm.at[pl.ds(er, CE)], src_vm)
        pltpu.sync_copy(w_hbm.at[pl.ds(er, CE)], w_vm)
        pltpu.sync_copy(dst_hbm.at[pl.ds(rr, CR)], dst_vm)

        def compute(i, cc):
            idx = src_vm[pl.ds(i * L, L)]
            hv = plsc.load_gather(h_vm, [idx])
            wv = w_vm[pl.ds(i * L, L)]
            m_vm[pl.ds(i * L, L)] = hv * wv
            return cc
        lax.fori_loop(0, CE // L, compute, 0)

        def scatter(j, cc):
            pltpu.sync_copy(m_vm.at[pl.ds(j * 128, 128)],
                            s_sh.at[dst_vm.at[j]], add=True)
            pltpu.sync_copy(ones_vm.at[0], cnt_sh.at[dst_vm.at[j]], add=True)
            return cc
        lax.fori_loop(0, CR, scatter, 0)
        return carry

    lax.fori_loop(0, chunks, chunk_body, 0)
    plsc.subcore_barrier()

    pltpu.sync_copy(s_sh.at[pl.ds(off, sl)], s_out.at[c, pl.ds(off, sl)])
    pltpu.sync_copy(cnt_sh.at[pl.ds(off, sl)], cnt_out.at[c, pl.ds(off, sl)])


def _combine_body(s_ref, c_ref, o_ref):
    stot = s_ref[0] + s_ref[1]
    ctot = jnp.maximum(c_ref[0] + c_ref[1], 1.0)
    o_ref[...] = stot / ctot


def kernel(x, w, src, dst, anchors):
    n = x.shape[0]
    e = src.shape[0]
    a = anchors.shape[0]
    arows = a // 128

    nacc = -(-(n + 256) // (128 * NS)) * (128 * NS)
    sl = nacc // NS
    rows = nacc // 128

    per_worker = NW * CE
    e_pad = -(-e // per_worker) * per_worker
    pad = e_pad - e
    rpw = e_pad // (NW * 128)
    chunks = e_pad // (NW * CE)

    src_p = jnp.concatenate([src, jnp.zeros((pad,), jnp.int32)])
    w_p = jnp.concatenate([w, jnp.zeros((pad,), jnp.float32)])
    dst_pad = n + (jnp.arange(pad, dtype=jnp.int32) % 256)
    dst_p = jnp.concatenate([dst, dst_pad]).reshape(e_pad // 128, 128)
    anch2 = anchors.reshape(arows, 128)
    ones2 = jnp.ones((1, 128), jnp.float32)
    zeros = jnp.zeros((nacc,), jnp.float32)

    mesh = plsc.VectorSubcoreMesh(core_axis_name="c", subcore_axis_name="s",
                                  num_cores=NC, num_subcores=NS)
    body = functools.partial(_agg_body, nacc=nacc, sl=sl, rpw=rpw,
                             chunks=chunks, arows=arows)
    agg = pl.kernel(
        body,
        out_type=[
            jax.ShapeDtypeStruct((NC, nacc), jnp.float32),
            jax.ShapeDtypeStruct((NC, nacc), jnp.float32),
            jax.ShapeDtypeStruct((NC, nacc), jnp.float32),
        ],
        mesh=mesh,
        scratch_types=[
            pltpu.VMEM((nacc,), jnp.float32),
            pltpu.VMEM((CE,), jnp.int32),
            pltpu.VMEM((CE,), jnp.float32),
            pltpu.VMEM((CR, 128), jnp.int32),
            pltpu.VMEM((CE,), jnp.float32),
            pltpu.VMEM((arows, 128), jnp.int32),
            pltpu.VMEM((1, 128), jnp.float32),
            pltpu.VMEM((1, 128), jnp.float32),
            pltpu.VMEM_SHARED((nacc,), jnp.float32),
            pltpu.VMEM_SHARED((nacc,), jnp.float32),
        ],
        compiler_params=pltpu.CompilerParams(needs_layout_passes=False),
    )
    s_part, cnt_part, _ = agg(x, w_p, src_p, dst_p, anch2, ones2, zeros)

    combine = pl.pallas_call(
        _combine_body,
        out_shape=jax.ShapeDtypeStruct((rows, 128), jnp.float32),
    )
    ho = combine(s_part.reshape(NC, rows, 128), cnt_part.reshape(NC, rows, 128))
    h_o = ho.reshape(-1)[:n]
    return (h_o, x)


# speedup vs baseline: 223.8957x; 2.8309x over previous
import functools

import jax
import jax.numpy as jnp
from jax import lax
from jax.experimental import pallas as pl
from jax.experimental.pallas import tpu as pltpu
from jax.experimental.pallas import tpu_sc as plsc

NC = 2
NS = 16
L = 16
NW = NC * NS
CE = 1024
CR = CE // 128
NBUF = 3


def _agg_body(x_hbm, w_hbm, src_hbm, dst_hbm, anch_hbm, ones_hbm, zeros_hbm,
              s_out, cnt_out, h_scr,
              h_vm, src_vm0, src_vm1, src_vm2, w_vm0, w_vm1, w_vm2,
              dst_vm0, dst_vm1, dst_vm2, m_vm0, m_vm1, m_vm2,
              anch_vm, xa_vm, ones_vm,
              s_sh, cnt_sh, sem_in, sem_sc,
              *, nacc, sl, rpw, chunks, arows):
    c = lax.axis_index("c")
    s = lax.axis_index("s")
    wid = s * NC + c

    off = s * sl
    pltpu.sync_copy(zeros_hbm.at[pl.ds(off, sl)], s_sh.at[pl.ds(off, sl)])
    pltpu.sync_copy(zeros_hbm.at[pl.ds(off, sl)], cnt_sh.at[pl.ds(off, sl)])
    pltpu.sync_copy(ones_hbm, ones_vm)
    plsc.subcore_barrier()

    @pl.when(s == 0)
    def _build_h():
        nblk = arows // 8

        def set_ones(blk, carry):
            pltpu.sync_copy(anch_hbm.at[pl.ds(blk * 8, 8)], anch_vm)

            def row(j, cc):
                pltpu.sync_copy(ones_vm.at[0], s_sh.at[anch_vm.at[j]])
                return cc
            lax.fori_loop(0, 8, row, 0)
            return carry
        lax.fori_loop(0, nblk, set_ones, 0)

        def add_x(blk, carry):
            pltpu.sync_copy(anch_hbm.at[pl.ds(blk * 8, 8)], anch_vm)

            def row(j, cc):
                pltpu.sync_copy(x_hbm.at[anch_vm.at[j]], xa_vm.at[0])
                pltpu.sync_copy(xa_vm.at[0], s_sh.at[anch_vm.at[j]], add=True)
                return cc
            lax.fori_loop(0, 8, row, 0)
            return carry
        lax.fori_loop(0, nblk, add_x, 0)

        pltpu.sync_copy(s_sh, h_scr.at[c])

    plsc.subcore_barrier()

    pltpu.sync_copy(zeros_hbm.at[pl.ds(off, sl)], s_sh.at[pl.ds(off, sl)])
    pltpu.sync_copy(h_scr.at[c], h_vm)
    plsc.subcore_barrier()

    ebase = wid * rpw * 128
    rbase = wid * rpw
    src_vms = (src_vm0, src_vm1, src_vm2)
    w_vms = (w_vm0, w_vm1, w_vm2)
    dst_vms = (dst_vm0, dst_vm1, dst_vm2)
    m_vms = (m_vm0, m_vm1, m_vm2)

    def fire_in(k, b):
        er = ebase + k * CE
        rr = rbase + k * CR
        pltpu.async_copy(src_hbm.at[pl.ds(er, CE)], src_vms[b], sem_in.at[b])
        pltpu.async_copy(w_hbm.at[pl.ds(er, CE)], w_vms[b], sem_in.at[b])
        pltpu.async_copy(dst_hbm.at[pl.ds(rr, CR)], dst_vms[b], sem_in.at[b])

    def wait_in(b):
        pltpu.make_async_copy(src_hbm.at[pl.ds(0, CE)], src_vms[b],
                              sem_in.at[b]).wait()
        pltpu.make_async_copy(w_hbm.at[pl.ds(0, CE)], w_vms[b],
                              sem_in.at[b]).wait()
        pltpu.make_async_copy(dst_hbm.at[pl.ds(0, CR)], dst_vms[b],
                              sem_in.at[b]).wait()

    def fire_sc(b):
        for j in range(CR):
            pltpu.async_copy(m_vms[b].at[pl.ds(j * 128, 128)],
                             s_sh.at[dst_vms[b].at[j]], sem_sc.at[b],
                             add=True)
            pltpu.async_copy(ones_vm.at[0],
                             cnt_sh.at[dst_vms[b].at[j]], sem_sc.at[b],
                             add=True)

    def drain_sc(b):
        for j in range(CR):
            pltpu.make_async_copy(m_vms[b].at[pl.ds(j * 128, 128)],
                                  s_sh.at[dst_vms[b].at[j]],
                                  sem_sc.at[b]).wait()
            pltpu.make_async_copy(ones_vm.at[0],
                                  cnt_sh.at[dst_vms[b].at[j]],
                                  sem_sc.at[b]).wait()

    fire_in(0, 0)

    def super_body(q, carry):
        k0 = q * NBUF
        for b in range(NBUF):
            k = k0 + b
            nb = (b + 1) % NBUF

            @pl.when(k >= 2)
            def _drain():
                drain_sc(nb)

            @pl.when(k + 1 < chunks)
            def _prefetch():
                fire_in(k + 1, nb)

            wait_in(b)

            src_b, w_b, m_b = src_vms[b], w_vms[b], m_vms[b]

            @plsc.parallel_loop(0, CE // L, unroll=4)
            def compute(i):
                idx = src_b[pl.ds(i * L, L)]
                hv = plsc.load_gather(h_vm, [idx])
                wv = w_b[pl.ds(i * L, L)]
                m_b[pl.ds(i * L, L)] = hv * wv

            fire_sc(b)
        return carry

    lax.fori_loop(0, chunks // NBUF, super_body, 0)
    drain_sc((chunks - 2) % NBUF)
    drain_sc((chunks - 1) % NBUF)
    plsc.subcore_barrier()

    pltpu.sync_copy(s_sh.at[pl.ds(off, sl)], s_out.at[c, pl.ds(off, sl)])
    pltpu.sync_copy(cnt_sh.at[pl.ds(off, sl)], cnt_out.at[c, pl.ds(off, sl)])


def _combine_body(s_ref, c_ref, o_ref):
    stot = s_ref[0] + s_ref[1]
    ctot = jnp.maximum(c_ref[0] + c_ref[1], 1.0)
    o_ref[...] = stot / ctot


def kernel(x, w, src, dst, anchors):
    n = x.shape[0]
    e = src.shape[0]
    a = anchors.shape[0]
    arows = a // 128

    nacc = -(-(n + 256) // (128 * NS)) * (128 * NS)
    sl = nacc // NS
    rows = nacc // 128

    per_super = NW * CE * NBUF
    e_pad = -(-e // per_super) * per_super
    pad = e_pad - e
    rpw = e_pad // (NW * 128)
    chunks = e_pad // (NW * CE)

    src_p = jnp.concatenate([src, jnp.zeros((pad,), jnp.int32)])
    w_p = jnp.concatenate([w, jnp.zeros((pad,), jnp.float32)])
    dst_pad = n + (jnp.arange(pad, dtype=jnp.int32) % 256)
    dst_p = jnp.concatenate([dst, dst_pad]).reshape(e_pad // 128, 128)
    anch2 = anchors.reshape(arows, 128)
    ones2 = jnp.ones((1, 128), jnp.float32)
    zeros = jnp.zeros((nacc,), jnp.float32)

    mesh = plsc.VectorSubcoreMesh(core_axis_name="c", subcore_axis_name="s",
                                  num_cores=NC, num_subcores=NS)
    body = functools.partial(_agg_body, nacc=nacc, sl=sl, rpw=rpw,
                             chunks=chunks, arows=arows)
    agg = pl.kernel(
        body,
        out_type=[
            jax.ShapeDtypeStruct((NC, nacc), jnp.float32),
            jax.ShapeDtypeStruct((NC, nacc), jnp.float32),
            jax.ShapeDtypeStruct((NC, nacc), jnp.float32),
        ],
        mesh=mesh,
        scratch_types=[
            pltpu.VMEM((nacc,), jnp.float32),
            pltpu.VMEM((CE,), jnp.int32),
            pltpu.VMEM((CE,), jnp.int32),
            pltpu.VMEM((CE,), jnp.int32),
            pltpu.VMEM((CE,), jnp.float32),
            pltpu.VMEM((CE,), jnp.float32),
            pltpu.VMEM((CE,), jnp.float32),
            pltpu.VMEM((CR, 128), jnp.int32),
            pltpu.VMEM((CR, 128), jnp.int32),
            pltpu.VMEM((CR, 128), jnp.int32),
            pltpu.VMEM((CE,), jnp.float32),
            pltpu.VMEM((CE,), jnp.float32),
            pltpu.VMEM((CE,), jnp.float32),
            pltpu.VMEM((8, 128), jnp.int32),
            pltpu.VMEM((1, 128), jnp.float32),
            pltpu.VMEM((1, 128), jnp.float32),
            pltpu.VMEM_SHARED((nacc,), jnp.float32),
            pltpu.VMEM_SHARED((nacc,), jnp.float32),
            pltpu.SemaphoreType.DMA((NBUF,)),
            pltpu.SemaphoreType.DMA((NBUF,)),
        ],
        compiler_params=pltpu.CompilerParams(needs_layout_passes=False),
    )
    s_part, cnt_part, _ = agg(x, w_p, src_p, dst_p, anch2, ones2, zeros)

    combine = pl.pallas_call(
        _combine_body,
        out_shape=jax.ShapeDtypeStruct((rows, 128), jnp.float32),
    )
    ho = combine(s_part.reshape(NC, rows, 128), cnt_part.reshape(NC, rows, 128))
    h_o = ho.reshape(-1)[:n]
    return (h_o, x)
